# baseline (speedup 1.0000x reference)
import functools

import jax
import jax.numpy as jnp
import numpy as np
from jax import lax
from jax.experimental import pallas as pl
from jax.experimental.pallas import tpu as pltpu
from jax.experimental.pallas import tpu_sc as plsc

_N = 512
_D_EMB = 64
_D_HID = 64

_NC = 2
_NS = 16
_NW = _NC * _NS
_L = 16
_RW = _N // _NW
_NJC = _N // _L

_consts = {}


def _threefry2x32_np(k0, k1, x0, x1):
    rot_a = (13, 15, 26, 6)
    rot_b = (17, 29, 16, 24)
    ks = [np.uint32(k0), np.uint32(k1),
          np.uint32(k0) ^ np.uint32(k1) ^ np.uint32(0x1BD11BDA)]
    x0 = (x0 + ks[0]).astype(np.uint32)
    x1 = (x1 + ks[1]).astype(np.uint32)

    def rotl(x, r):
        return ((x << np.uint32(r)) | (x >> np.uint32(32 - r))).astype(np.uint32)

    for g, rots in enumerate((rot_a, rot_b, rot_a, rot_b, rot_a)):
        for r in rots:
            x0 = (x0 + x1).astype(np.uint32)
            x1 = x0 ^ rotl(x1, r)
        x0 = (x0 + ks[(g + 1) % 3]).astype(np.uint32)
        x1 = (x1 + ks[(g + 2) % 3] + np.uint32(g + 1)).astype(np.uint32)
    return x0, x1


def _noise_logit_np():
    if "nl" not in _consts:
        n = _N * _N
        b0, b1_ = _threefry2x32_np(
            0, 42, np.zeros(n, dtype=np.uint32), np.arange(n, dtype=np.uint32)
        )
        bits = b0 ^ b1_
        fb = (bits >> np.uint32(9)) | np.uint32(0x3F800000)
        floats = fb.view(np.float32) - np.float32(1.0)
        mn = np.float32(1e-6)
        mx = np.float32(1.0 - 1e-6)
        u = np.maximum(mn, floats * (mx - mn) + mn)
        nl = (np.log(u) - np.log(np.float32(1.0) - u)).astype(np.float32)
        nl = nl.reshape(_N, _N)
        _consts["nl"] = nl
        _consts["nlT"] = np.ascontiguousarray(nl.T)
    return _consts["nl"], _consts["nlT"]


def _prep_body(embed_ref, w1_ref, b1_ref, ab_ref, bb_ref, abt_ref, bbt_ref):
    w1a = w1_ref[:_D_EMB, :]
    w1b = w1_ref[_D_EMB:, :]
    ab = (
        jnp.dot(embed_ref[...], w1a, preferred_element_type=jnp.float32)
        + b1_ref[...]
    )
    bb = jnp.dot(embed_ref[...], w1b, preferred_element_type=jnp.float32)
    ab_ref[...] = ab
    bb_ref[...] = bb
    abt_ref[...] = ab.T
    bbt_ref[...] = bb.T


def _prep(embed, W1, b1r):
    return pl.pallas_call(
        _prep_body,
        out_shape=[
            jax.ShapeDtypeStruct((_N, _D_HID), jnp.float32),
            jax.ShapeDtypeStruct((_N, _D_HID), jnp.float32),
            jax.ShapeDtypeStruct((_D_HID, _N), jnp.float32),
            jax.ShapeDtypeStruct((_D_HID, _N), jnp.float32),
        ],
    )(embed, W1, b1r)


@functools.partial(
    pl.kernel,
    out_type=jax.ShapeDtypeStruct((_N, _N), jnp.float32),
    mesh=plsc.VectorSubcoreMesh(core_axis_name="c", subcore_axis_name="s"),
    compiler_params=pltpu.CompilerParams(needs_layout_passes=False),
    scratch_types=[
        pltpu.VMEM((_RW, _D_HID), jnp.float32),
        pltpu.VMEM((_RW, _D_HID), jnp.float32),
        pltpu.VMEM((_D_HID, _N), jnp.float32),
        pltpu.VMEM((_D_HID, _N), jnp.float32),
        pltpu.VMEM((_D_HID,), jnp.float32),
        pltpu.VMEM((_L,), jnp.float32),
        pltpu.VMEM((_L,), jnp.float32),
        pltpu.VMEM((_RW, _N), jnp.float32),
        pltpu.VMEM((_RW, _N), jnp.float32),
        pltpu.VMEM((_RW, _N), jnp.float32),
        pltpu.VMEM((_RW, _N), jnp.float32),
        pltpu.VMEM((_RW, _N), jnp.float32),
    ],
)
def _sc_pair(
    ab_hbm, bb_hbm, abt_hbm, bbt_hbm, w2_hbm, itmp_hbm, b2v_hbm,
    adj_hbm, nl_hbm, nlt_hbm, out_hbm,
    ab_v, bb_v, abt_v, bbt_v, w2_v, itmp_v, b2v_v,
    adj_v, nl_v, nlt_v, v1_v, v2_v,
):
    wid = lax.axis_index("s") * _NC + lax.axis_index("c")
    base = wid * _RW

    pltpu.sync_copy(ab_hbm.at[pl.ds(base, _RW)], ab_v)
    pltpu.sync_copy(bb_hbm.at[pl.ds(base, _RW)], bb_v)
    pltpu.sync_copy(abt_hbm, abt_v)
    pltpu.sync_copy(bbt_hbm, bbt_v)
    pltpu.sync_copy(w2_hbm, w2_v)
    pltpu.sync_copy(itmp_hbm, itmp_v)
    pltpu.sync_copy(b2v_hbm, b2v_v)
    pltpu.sync_copy(adj_hbm.at[pl.ds(base, _RW)], adj_v)
    pltpu.sync_copy(nl_hbm.at[pl.ds(base, _RW)], nl_v)
    pltpu.sync_copy(nlt_hbm.at[pl.ds(base, _RW)], nlt_v)

    zero = jnp.zeros((_L,), jnp.float32)

    def zbody(r, c):
        for jc in range(_NJC):
            v1_v[r, pl.ds(jc * _L, _L)] = zero
            v2_v[r, pl.ds(jc * _L, _L)] = zero
        return c

    lax.fori_loop(0, _RW, zbody, 0)

    def accumulate(bt_ref, a_ref, acc_ref):
        def kbody(k, c):
            kv = jnp.full((_L,), k, jnp.int32)
            sw = plsc.load_gather(w2_v, [kv])
            bt = [bt_ref[k, pl.ds(jc * _L, _L)] for jc in range(_NJC)]

            def rbody(r, c2):
                rv = jnp.full((_L,), r, jnp.int32)
                sa = plsc.load_gather(a_ref, [rv, kv])
                for jc in range(_NJC):
                    t = jnp.maximum(bt[jc] + sa, 0.0) * sw
                    plsc.addupdate(acc_ref.at[r, pl.ds(jc * _L, _L)], t)
                return c2

            return lax.fori_loop(0, _RW, rbody, c)

        lax.fori_loop(0, _D_HID, kbody, 0)

    accumulate(bbt_v, ab_v, v1_v)
    accumulate(abt_v, bb_v, v2_v)

    itmp = itmp_v[...]
    b2v = b2v_v[...]
    half = jnp.full((_L,), 0.5, jnp.float32)
    one = jnp.full((_L,), 1.0, jnp.float32)

    def ebody(r, c):
        for jc in range(_NJC):
            d = pl.ds(jc * _L, _L)
            x1 = (nl_v[r, d] + v1_v[r, d] + b2v) * itmp
            g1 = one / (one + jnp.exp(-x1))
            x2 = (nlt_v[r, d] + v2_v[r, d] + b2v) * itmp
            g2 = one / (one + jnp.exp(-x2))
            v1_v[r, d] = adj_v[r, d] * (half * (g1 + g2))
        return c

    lax.fori_loop(0, _RW, ebody, 0)

    pltpu.sync_copy(v1_v, out_hbm.at[pl.ds(base, _RW)])


def kernel(x, embed, adj, W1, b1, W2, b2, tmp, label, sub_nodes):
    del x, label, sub_nodes
    nl_np, nlt_np = _noise_logit_np()
    nl = jnp.asarray(nl_np)
    nlt = jnp.asarray(nlt_np)
    b1r = b1.reshape(1, _D_HID)

    ab, bb, abt, bbt = _prep(embed, W1, b1r)

    w2f = W2.reshape(_D_HID)
    tmpf = jnp.asarray(tmp, jnp.float32)
    itmp16 = jnp.full((_L,), 1.0, jnp.float32) / tmpf
    b2v16 = jnp.broadcast_to(b2.astype(jnp.float32), (_L,))

    return _sc_pair(ab, bb, abt, bbt, w2f, itmp16, b2v16, adj, nl, nlt)


# speedup vs baseline: 1.3612x; 1.3612x over previous
import functools

import jax
import jax.numpy as jnp
import numpy as np
from jax import lax
from jax.experimental import pallas as pl
from jax.experimental.pallas import tpu as pltpu
from jax.experimental.pallas import tpu_sc as plsc

_N = 512
_D_EMB = 64
_D_HID = 64

_NC = 2
_NS = 16
_NW = _NC * _NS
_L = 16

_SC_ROWS = 256
_TC_ROWS = _N - _SC_ROWS
_RW = _SC_ROWS // _NW
_NJC = _N // _L
_BI = 64
_TC_ROW0 = _SC_ROWS // _BI

_consts = {}


def _threefry2x32_np(k0, k1, x0, x1):
    rot_a = (13, 15, 26, 6)
    rot_b = (17, 29, 16, 24)
    ks = [np.uint32(k0), np.uint32(k1),
          np.uint32(k0) ^ np.uint32(k1) ^ np.uint32(0x1BD11BDA)]
    x0 = (x0 + ks[0]).astype(np.uint32)
    x1 = (x1 + ks[1]).astype(np.uint32)

    def rotl(x, r):
        return ((x << np.uint32(r)) | (x >> np.uint32(32 - r))).astype(np.uint32)

    for g, rots in enumerate((rot_a, rot_b, rot_a, rot_b, rot_a)):
        for r in rots:
            x0 = (x0 + x1).astype(np.uint32)
            x1 = x0 ^ rotl(x1, r)
        x0 = (x0 + ks[(g + 1) % 3]).astype(np.uint32)
        x1 = (x1 + ks[(g + 2) % 3] + np.uint32(g + 1)).astype(np.uint32)
    return x0, x1


def _noise_logit_np():
    if "nl" not in _consts:
        n = _N * _N
        b0, b1_ = _threefry2x32_np(
            0, 42, np.zeros(n, dtype=np.uint32), np.arange(n, dtype=np.uint32)
        )
        bits = b0 ^ b1_
        fb = (bits >> np.uint32(9)) | np.uint32(0x3F800000)
        floats = fb.view(np.float32) - np.float32(1.0)
        mn = np.float32(1e-6)
        mx = np.float32(1.0 - 1e-6)
        u = np.maximum(mn, floats * (mx - mn) + mn)
        nl = (np.log(u) - np.log(np.float32(1.0) - u)).astype(np.float32)
        nl = nl.reshape(_N, _N)
        _consts["nl"] = nl
        _consts["nlT"] = np.ascontiguousarray(nl.T)
    return _consts["nl"], _consts["nlT"]


def _prep_body(embed_ref, w1_ref, b1_ref, ab_ref, bb_ref, abt_ref, bbt_ref):
    w1a = w1_ref[:_D_EMB, :]
    w1b = w1_ref[_D_EMB:, :]
    ab = (
        jnp.dot(embed_ref[...], w1a, preferred_element_type=jnp.float32)
        + b1_ref[...]
    )
    bb = jnp.dot(embed_ref[...], w1b, preferred_element_type=jnp.float32)
    ab_ref[...] = ab
    bb_ref[...] = bb
    abt_ref[...] = ab.T
    bbt_ref[...] = bb.T


def _prep(embed, W1, b1r):
    return pl.pallas_call(
        _prep_body,
        out_shape=[
            jax.ShapeDtypeStruct((_N, _D_HID), jnp.float32),
            jax.ShapeDtypeStruct((_N, _D_HID), jnp.float32),
            jax.ShapeDtypeStruct((_D_HID, _N), jnp.float32),
            jax.ShapeDtypeStruct((_D_HID, _N), jnp.float32),
        ],
    )(embed, W1, b1r)


@functools.partial(
    pl.kernel,
    out_type=jax.ShapeDtypeStruct((_SC_ROWS, _N), jnp.float32),
    mesh=plsc.VectorSubcoreMesh(core_axis_name="c", subcore_axis_name="s"),
    compiler_params=pltpu.CompilerParams(needs_layout_passes=False),
    scratch_types=[
        pltpu.VMEM((_RW, _D_HID), jnp.float32),
        pltpu.VMEM((_RW, _D_HID), jnp.float32),
        pltpu.VMEM((_D_HID, _N), jnp.float32),
        pltpu.VMEM((_D_HID, _N), jnp.float32),
        pltpu.VMEM((_D_HID,), jnp.float32),
        pltpu.VMEM((_L,), jnp.float32),
        pltpu.VMEM((_L,), jnp.float32),
        pltpu.VMEM((_RW, _N), jnp.float32),
        pltpu.VMEM((_RW, _N), jnp.float32),
        pltpu.VMEM((_RW, _N), jnp.float32),
        pltpu.VMEM((_RW, _N), jnp.float32),
        pltpu.VMEM((_RW, _N), jnp.float32),
    ],
)
def _sc_pair(
    ab_hbm, bb_hbm, abt_hbm, bbt_hbm, w2_hbm, itmp_hbm, b2v_hbm,
    adj_hbm, nl_hbm, nlt_hbm, out_hbm,
    ab_v, bb_v, abt_v, bbt_v, w2_v, itmp_v, b2v_v,
    adj_v, nl_v, nlt_v, v1_v, v2_v,
):
    wid = lax.axis_index("s") * _NC + lax.axis_index("c")
    base = wid * _RW

    pltpu.sync_copy(ab_hbm.at[pl.ds(base, _RW)], ab_v)
    pltpu.sync_copy(bb_hbm.at[pl.ds(base, _RW)], bb_v)
    pltpu.sync_copy(abt_hbm, abt_v)
    pltpu.sync_copy(bbt_hbm, bbt_v)
    pltpu.sync_copy(w2_hbm, w2_v)
    pltpu.sync_copy(itmp_hbm, itmp_v)
    pltpu.sync_copy(b2v_hbm, b2v_v)
    pltpu.sync_copy(adj_hbm.at[pl.ds(base, _RW)], adj_v)
    pltpu.sync_copy(nl_hbm.at[pl.ds(base, _RW)], nl_v)
    pltpu.sync_copy(nlt_hbm.at[pl.ds(base, _RW)], nlt_v)

    zero = jnp.zeros((_L,), jnp.float32)

    def zbody(r, c):
        for jc in range(_NJC):
            v1_v[r, pl.ds(jc * _L, _L)] = zero
            v2_v[r, pl.ds(jc * _L, _L)] = zero
        return c

    lax.fori_loop(0, _RW, zbody, 0)

    def accumulate(bt_ref, a_ref, acc_ref):
        def kbody(k, c):
            kv = jnp.full((_L,), k, jnp.int32)
            sw = plsc.load_gather(w2_v, [kv])
            bt = [bt_ref[k, pl.ds(jc * _L, _L)] for jc in range(_NJC)]

            def rbody(r, c2):
                rv = jnp.full((_L,), r, jnp.int32)
                sa = plsc.load_gather(a_ref, [rv, kv])
                for jc in range(_NJC):
                    t = jnp.maximum(bt[jc] + sa, 0.0) * sw
                    plsc.addupdate(acc_ref.at[r, pl.ds(jc * _L, _L)], t)
                return c2

            return lax.fori_loop(0, _RW, rbody, c)

        lax.fori_loop(0, _D_HID, kbody, 0)

    accumulate(bbt_v, ab_v, v1_v)
    accumulate(abt_v, bb_v, v2_v)

    itmp = itmp_v[...]
    b2v = b2v_v[...]
    half = jnp.full((_L,), 0.5, jnp.float32)
    one = jnp.full((_L,), 1.0, jnp.float32)

    def ebody(r, c):
        for jc in range(_NJC):
            d = pl.ds(jc * _L, _L)
            x1 = (nl_v[r, d] + v1_v[r, d] + b2v) * itmp
            g1 = one / (one + jnp.exp(-x1))
            x2 = (nlt_v[r, d] + v2_v[r, d] + b2v) * itmp
            g2 = one / (one + jnp.exp(-x2))
            v1_v[r, d] = adj_v[r, d] * (half * (g1 + g2))
        return c

    lax.fori_loop(0, _RW, ebody, 0)

    pltpu.sync_copy(v1_v, out_hbm.at[pl.ds(base, _RW)])


def _tc_pair_body(
    embed_ref, eblk_ref, w1_ref, b1_ref, w2_ref, b2_ref, tmp_ref,
    adj_ref, nl_ref, nlt_ref, out_ref, a_scr, b_scr,
):
    i = pl.program_id(0)
    w1a = w1_ref[:_D_EMB, :]
    w1b = w1_ref[_D_EMB:, :]

    @pl.when(i == 0)
    def _init():
        a_scr[...] = (
            jnp.dot(embed_ref[...], w1a, preferred_element_type=jnp.float32)
            + b1_ref[...]
        )
        b_scr[...] = jnp.dot(embed_ref[...], w1b, preferred_element_type=jnp.float32)

    a_i = (
        jnp.dot(eblk_ref[...], w1a, preferred_element_type=jnp.float32)
        + b1_ref[...]
    )
    b_i = jnp.dot(eblk_ref[...], w1b, preferred_element_type=jnp.float32)

    itmp = 1.0 / tmp_ref[0, 0]
    b2 = b2_ref[0, 0]

    t1 = jnp.maximum(a_i[:, None, :] + b_scr[...][None, :, :], 0.0)
    v1 = jnp.dot(
        t1.reshape(_BI * _N, _D_HID), w2_ref[...], preferred_element_type=jnp.float32
    ).reshape(_BI, _N)
    t2 = jnp.maximum(b_i[:, None, :] + a_scr[...][None, :, :], 0.0)
    v2 = jnp.dot(
        t2.reshape(_BI * _N, _D_HID), w2_ref[...], preferred_element_type=jnp.float32
    ).reshape(_BI, _N)

    g1 = jax.nn.sigmoid((nl_ref[...] + v1 + b2) * itmp)
    g2 = jax.nn.sigmoid((nlt_ref[...] + v2 + b2) * itmp)
    out_ref[...] = adj_ref[...] * (0.5 * (g1 + g2))


def _tc_pair(embed, W1, b1r, W2, b2r, tmpr, adj, nl, nlt):
    grid = (_TC_ROWS // _BI,)
    return pl.pallas_call(
        _tc_pair_body,
        grid=grid,
        in_specs=[
            pl.BlockSpec((_N, _D_EMB), lambda i: (0, 0)),
            pl.BlockSpec((_BI, _D_EMB), lambda i: (i + _TC_ROW0, 0)),
            pl.BlockSpec((2 * _D_EMB, _D_HID), lambda i: (0, 0)),
            pl.BlockSpec((1, _D_HID), lambda i: (0, 0)),
            pl.BlockSpec((_D_HID, 1), lambda i: (0, 0)),
            pl.BlockSpec((1, 1), lambda i: (0, 0)),
            pl.BlockSpec((1, 1), lambda i: (0, 0)),
            pl.BlockSpec((_BI, _N), lambda i: (i + _TC_ROW0, 0)),
            pl.BlockSpec((_BI, _N), lambda i: (i + _TC_ROW0, 0)),
            pl.BlockSpec((_BI, _N), lambda i: (i + _TC_ROW0, 0)),
        ],
        out_specs=pl.BlockSpec((_BI, _N), lambda i: (i, 0)),
        out_shape=jax.ShapeDtypeStruct((_TC_ROWS, _N), jnp.float32),
        scratch_shapes=[
            pltpu.VMEM((_N, _D_HID), jnp.float32),
            pltpu.VMEM((_N, _D_HID), jnp.float32),
        ],
    )(embed, embed, W1, b1r, W2, b2r, tmpr, adj, nl, nlt)


def kernel(x, embed, adj, W1, b1, W2, b2, tmp, label, sub_nodes):
    del x, label, sub_nodes
    nl_np, nlt_np = _noise_logit_np()
    nl = jnp.asarray(nl_np)
    nlt = jnp.asarray(nlt_np)
    b1r = b1.reshape(1, _D_HID)
    b2r = jnp.asarray(b2, jnp.float32).reshape(1, 1)
    tmpr = jnp.asarray(tmp, jnp.float32).reshape(1, 1)

    ab, bb, abt, bbt = _prep(embed, W1, b1r)

    w2f = W2.reshape(_D_HID)
    tmpf = jnp.asarray(tmp, jnp.float32)
    itmp16 = jnp.full((_L,), 1.0, jnp.float32) / tmpf
    b2v16 = jnp.broadcast_to(b2.astype(jnp.float32), (_L,))

    out_sc = _sc_pair(ab, bb, abt, bbt, w2f, itmp16, b2v16, adj, nl, nlt)
    out_tc = _tc_pair(embed, W1, b1r, W2, b2r, tmpr, adj, nl, nlt)
    return jnp.concatenate([out_sc, out_tc], axis=0)


# speedup vs baseline: 1.4085x; 1.0347x over previous
import functools

import jax
import jax.numpy as jnp
import numpy as np
from jax import lax
from jax.experimental import pallas as pl
from jax.experimental.pallas import tpu as pltpu
from jax.experimental.pallas import tpu_sc as plsc

_N = 512
_D_EMB = 64
_D_HID = 64

_NC = 2
_NS = 16
_NW = _NC * _NS
_L = 16

_SC_ROWS = 256
_TC_ROWS = _N - _SC_ROWS
_RW = _SC_ROWS // _NW
_NJC = _N // _L
_BI = 64
_TC_ROW0 = _SC_ROWS // _BI

_consts = {}


def _threefry2x32_np(k0, k1, x0, x1):
    rot_a = (13, 15, 26, 6)
    rot_b = (17, 29, 16, 24)
    ks = [np.uint32(k0), np.uint32(k1),
          np.uint32(k0) ^ np.uint32(k1) ^ np.uint32(0x1BD11BDA)]
    x0 = (x0 + ks[0]).astype(np.uint32)
    x1 = (x1 + ks[1]).astype(np.uint32)

    def rotl(x, r):
        return ((x << np.uint32(r)) | (x >> np.uint32(32 - r))).astype(np.uint32)

    for g, rots in enumerate((rot_a, rot_b, rot_a, rot_b, rot_a)):
        for r in rots:
            x0 = (x0 + x1).astype(np.uint32)
            x1 = x0 ^ rotl(x1, r)
        x0 = (x0 + ks[(g + 1) % 3]).astype(np.uint32)
        x1 = (x1 + ks[(g + 2) % 3] + np.uint32(g + 1)).astype(np.uint32)
    return x0, x1


def _noise_logit_np():
    if "nl" not in _consts:
        n = _N * _N
        b0, b1_ = _threefry2x32_np(
            0, 42, np.zeros(n, dtype=np.uint32), np.arange(n, dtype=np.uint32)
        )
        bits = b0 ^ b1_
        fb = (bits >> np.uint32(9)) | np.uint32(0x3F800000)
        floats = fb.view(np.float32) - np.float32(1.0)
        mn = np.float32(1e-6)
        mx = np.float32(1.0 - 1e-6)
        u = np.maximum(mn, floats * (mx - mn) + mn)
        nl = (np.log(u) - np.log(np.float32(1.0) - u)).astype(np.float32)
        nl = nl.reshape(_N, _N)
        _consts["nl"] = nl
        _consts["nlT"] = np.ascontiguousarray(nl.T)
    return _consts["nl"], _consts["nlT"]


def _prep_body(
    embed_ref, w1_ref, b1_ref, b2_ref, tmp_ref,
    ab_ref, bb_ref, abt_ref, bbt_ref, aux_ref,
):
    w1a = w1_ref[:_D_EMB, :]
    w1b = w1_ref[_D_EMB:, :]
    ab = (
        jnp.dot(embed_ref[...], w1a, preferred_element_type=jnp.float32)
        + b1_ref[...]
    )
    bb = jnp.dot(embed_ref[...], w1b, preferred_element_type=jnp.float32)
    ab_ref[...] = ab[:_SC_ROWS, :]
    bb_ref[...] = bb[:_SC_ROWS, :]
    abt_ref[...] = ab.T
    bbt_ref[...] = bb.T
    aux_ref[0:1, :] = jnp.full((1, _L), 1.0, jnp.float32) / tmp_ref[0, 0]
    aux_ref[1:2, :] = jnp.full((1, _L), 1.0, jnp.float32) * b2_ref[0, 0]


def _prep(embed, W1, b1r, b2r, tmpr):
    return pl.pallas_call(
        _prep_body,
        out_shape=[
            jax.ShapeDtypeStruct((_SC_ROWS, _D_HID), jnp.float32),
            jax.ShapeDtypeStruct((_SC_ROWS, _D_HID), jnp.float32),
            jax.ShapeDtypeStruct((_D_HID, _N), jnp.float32),
            jax.ShapeDtypeStruct((_D_HID, _N), jnp.float32),
            jax.ShapeDtypeStruct((2, _L), jnp.float32),
        ],
    )(embed, W1, b1r, b2r, tmpr)


@functools.partial(
    pl.kernel,
    out_type=jax.ShapeDtypeStruct((_SC_ROWS, _N), jnp.float32),
    mesh=plsc.VectorSubcoreMesh(core_axis_name="c", subcore_axis_name="s"),
    compiler_params=pltpu.CompilerParams(needs_layout_passes=False),
    scratch_types=[
        pltpu.VMEM((_RW, _D_HID), jnp.float32),
        pltpu.VMEM((_RW, _D_HID), jnp.float32),
        pltpu.VMEM((_D_HID, _N), jnp.float32),
        pltpu.VMEM((_D_HID, _N), jnp.float32),
        pltpu.VMEM((_D_HID,), jnp.float32),
        pltpu.VMEM((2, _L), jnp.float32),
        pltpu.VMEM((_RW, _N), jnp.float32),
        pltpu.VMEM((_RW, _N), jnp.float32),
        pltpu.VMEM((_RW, _N), jnp.float32),
        pltpu.VMEM((_RW, _N), jnp.float32),
        pltpu.VMEM((_RW, _N), jnp.float32),
    ],
)
def _sc_pair(
    ab_hbm, bb_hbm, abt_hbm, bbt_hbm, w2_hbm, aux_hbm,
    adj_hbm, nl_hbm, nlt_hbm, out_hbm,
    ab_v, bb_v, abt_v, bbt_v, w2_v, aux_v,
    adj_v, nl_v, nlt_v, v1_v, v2_v,
):
    wid = lax.axis_index("s") * _NC + lax.axis_index("c")
    base = wid * _RW

    pltpu.sync_copy(ab_hbm.at[pl.ds(base, _RW)], ab_v)
    pltpu.sync_copy(bb_hbm.at[pl.ds(base, _RW)], bb_v)
    pltpu.sync_copy(abt_hbm, abt_v)
    pltpu.sync_copy(bbt_hbm, bbt_v)
    pltpu.sync_copy(w2_hbm, w2_v)
    pltpu.sync_copy(aux_hbm, aux_v)
    pltpu.sync_copy(adj_hbm.at[pl.ds(base, _RW)], adj_v)
    pltpu.sync_copy(nl_hbm.at[pl.ds(base, _RW)], nl_v)
    pltpu.sync_copy(nlt_hbm.at[pl.ds(base, _RW)], nlt_v)

    zero = jnp.zeros((_L,), jnp.float32)

    def zbody(r, c):
        for jc in range(_NJC):
            v1_v[r, pl.ds(jc * _L, _L)] = zero
            v2_v[r, pl.ds(jc * _L, _L)] = zero
        return c

    lax.fori_loop(0, _RW, zbody, 0)

    def accumulate(bt_ref, a_ref, acc_ref):
        def kbody(k, c):
            kv = jnp.full((_L,), k, jnp.int32)
            sw = plsc.load_gather(w2_v, [kv])
            bt = [bt_ref[k, pl.ds(jc * _L, _L)] for jc in range(_NJC)]

            def rbody(r, c2):
                rv = jnp.full((_L,), r, jnp.int32)
                sa = plsc.load_gather(a_ref, [rv, kv])
                for jc in range(_NJC):
                    t = jnp.maximum(bt[jc] + sa, 0.0) * sw
                    plsc.addupdate(acc_ref.at[r, pl.ds(jc * _L, _L)], t)
                return c2

            return lax.fori_loop(0, _RW, rbody, c)

        lax.fori_loop(0, _D_HID, kbody, 0)

    accumulate(bbt_v, ab_v, v1_v)
    accumulate(abt_v, bb_v, v2_v)

    itmp = aux_v[0, pl.ds(0, _L)]
    b2v = aux_v[1, pl.ds(0, _L)]
    half = jnp.full((_L,), 0.5, jnp.float32)
    one = jnp.full((_L,), 1.0, jnp.float32)

    def ebody(r, c):
        for jc in range(_NJC):
            d = pl.ds(jc * _L, _L)
            x1 = (nl_v[r, d] + v1_v[r, d] + b2v) * itmp
            g1 = one / (one + jnp.exp(-x1))
            x2 = (nlt_v[r, d] + v2_v[r, d] + b2v) * itmp
            g2 = one / (one + jnp.exp(-x2))
            v1_v[r, d] = adj_v[r, d] * (half * (g1 + g2))
        return c

    lax.fori_loop(0, _RW, ebody, 0)

    pltpu.sync_copy(v1_v, out_hbm.at[pl.ds(base, _RW)])


def _tc_pair_body(
    embed_ref, eblk_ref, w1_ref, b1_ref, w2_ref, b2_ref, tmp_ref,
    adj_ref, nl_ref, nlt_ref, out_ref, a_scr, b_scr,
):
    i = pl.program_id(0)
    w1a = w1_ref[:_D_EMB, :]
    w1b = w1_ref[_D_EMB:, :]

    @pl.when(i == 0)
    def _init():
        a_scr[...] = (
            jnp.dot(embed_ref[...], w1a, preferred_element_type=jnp.float32)
            + b1_ref[...]
        )
        b_scr[...] = jnp.dot(embed_ref[...], w1b, preferred_element_type=jnp.float32)

    a_i = (
        jnp.dot(eblk_ref[...], w1a, preferred_element_type=jnp.float32)
        + b1_ref[...]
    )
    b_i = jnp.dot(eblk_ref[...], w1b, preferred_element_type=jnp.float32)

    itmp = 1.0 / tmp_ref[0, 0]
    b2 = b2_ref[0, 0]

    t1 = jnp.maximum(a_i[:, None, :] + b_scr[...][None, :, :], 0.0)
    v1 = jnp.dot(
        t1.reshape(_BI * _N, _D_HID), w2_ref[...], preferred_element_type=jnp.float32
    ).reshape(_BI, _N)
    t2 = jnp.maximum(b_i[:, None, :] + a_scr[...][None, :, :], 0.0)
    v2 = jnp.dot(
        t2.reshape(_BI * _N, _D_HID), w2_ref[...], preferred_element_type=jnp.float32
    ).reshape(_BI, _N)

    g1 = jax.nn.sigmoid((nl_ref[...] + v1 + b2) * itmp)
    g2 = jax.nn.sigmoid((nlt_ref[...] + v2 + b2) * itmp)
    out_ref[...] = adj_ref[...] * (0.5 * (g1 + g2))


def _tc_pair(embed, W1, b1r, W2, b2r, tmpr, adj, nl, nlt):
    grid = (_TC_ROWS // _BI,)
    return pl.pallas_call(
        _tc_pair_body,
        grid=grid,
        in_specs=[
            pl.BlockSpec((_N, _D_EMB), lambda i: (0, 0)),
            pl.BlockSpec((_BI, _D_EMB), lambda i: (i + _TC_ROW0, 0)),
            pl.BlockSpec((2 * _D_EMB, _D_HID), lambda i: (0, 0)),
            pl.BlockSpec((1, _D_HID), lambda i: (0, 0)),
            pl.BlockSpec((_D_HID, 1), lambda i: (0, 0)),
            pl.BlockSpec((1, 1), lambda i: (0, 0)),
            pl.BlockSpec((1, 1), lambda i: (0, 0)),
            pl.BlockSpec((_BI, _N), lambda i: (i + _TC_ROW0, 0)),
            pl.BlockSpec((_BI, _N), lambda i: (i + _TC_ROW0, 0)),
            pl.BlockSpec((_BI, _N), lambda i: (i + _TC_ROW0, 0)),
        ],
        out_specs=pl.BlockSpec((_BI, _N), lambda i: (i, 0)),
        out_shape=jax.ShapeDtypeStruct((_TC_ROWS, _N), jnp.float32),
        scratch_shapes=[
            pltpu.VMEM((_N, _D_HID), jnp.float32),
            pltpu.VMEM((_N, _D_HID), jnp.float32),
        ],
    )(embed, embed, W1, b1r, W2, b2r, tmpr, adj, nl, nlt)


def kernel(x, embed, adj, W1, b1, W2, b2, tmp, label, sub_nodes):
    del x, label, sub_nodes
    nl_np, nlt_np = _noise_logit_np()
    nl = jnp.asarray(nl_np)
    nlt = jnp.asarray(nlt_np)
    b1r = b1.reshape(1, _D_HID)
    b2r = jnp.asarray(b2, jnp.float32).reshape(1, 1)
    tmpr = jnp.asarray(tmp, jnp.float32).reshape(1, 1)

    nl_sc = jnp.asarray(nl_np[:_SC_ROWS])
    nlt_sc = jnp.asarray(nlt_np[:_SC_ROWS])

    ab, bb, abt, bbt, aux = _prep(embed, W1, b1r, b2r, tmpr)

    w2f = W2.reshape(_D_HID)

    out_sc = _sc_pair(ab, bb, abt, bbt, w2f, aux, adj, nl_sc, nlt_sc)
    out_tc = _tc_pair(embed, W1, b1r, W2, b2r, tmpr, adj, nl, nlt)
    return jnp.concatenate([out_sc, out_tc], axis=0)


# speedup vs baseline: 1.7511x; 1.2433x over previous
import functools

import jax
import jax.numpy as jnp
import numpy as np
from jax import lax
from jax.experimental import pallas as pl
from jax.experimental.pallas import tpu as pltpu
from jax.experimental.pallas import tpu_sc as plsc

_N = 512
_D_EMB = 64
_D_HID = 64

_NC = 2
_NS = 16
_NW = _NC * _NS
_L = 16

_SC_ROWS = 128
_TC_ROWS = _N - _SC_ROWS
_RW = _SC_ROWS // _NW
_NJC = _N // _L
_BI = 64
_TC_ROW0 = _SC_ROWS // _BI

_consts = {}


def _threefry2x32_np(k0, k1, x0, x1):
    rot_a = (13, 15, 26, 6)
    rot_b = (17, 29, 16, 24)
    ks = [np.uint32(k0), np.uint32(k1),
          np.uint32(k0) ^ np.uint32(k1) ^ np.uint32(0x1BD11BDA)]
    x0 = (x0 + ks[0]).astype(np.uint32)
    x1 = (x1 + ks[1]).astype(np.uint32)

    def rotl(x, r):
        return ((x << np.uint32(r)) | (x >> np.uint32(32 - r))).astype(np.uint32)

    for g, rots in enumerate((rot_a, rot_b, rot_a, rot_b, rot_a)):
        for r in rots:
            x0 = (x0 + x1).astype(np.uint32)
            x1 = x0 ^ rotl(x1, r)
        x0 = (x0 + ks[(g + 1) % 3]).astype(np.uint32)
        x1 = (x1 + ks[(g + 2) % 3] + np.uint32(g + 1)).astype(np.uint32)
    return x0, x1


def _noise_logit_np():
    if "nl" not in _consts:
        n = _N * _N
        b0, b1_ = _threefry2x32_np(
            0, 42, np.zeros(n, dtype=np.uint32), np.arange(n, dtype=np.uint32)
        )
        bits = b0 ^ b1_
        fb = (bits >> np.uint32(9)) | np.uint32(0x3F800000)
        floats = fb.view(np.float32) - np.float32(1.0)
        mn = np.float32(1e-6)
        mx = np.float32(1.0 - 1e-6)
        u = np.maximum(mn, floats * (mx - mn) + mn)
        nl = (np.log(u) - np.log(np.float32(1.0) - u)).astype(np.float32)
        nl = nl.reshape(_N, _N)
        _consts["nl"] = nl
        _consts["nlT"] = np.ascontiguousarray(nl.T)
    return _consts["nl"], _consts["nlT"]


def _prep_body(
    embed_ref, w1_ref, b1_ref, b2_ref, tmp_ref,
    ab_ref, bb_ref, abt_ref, bbt_ref, aux_ref,
):
    w1a = w1_ref[:_D_EMB, :]
    w1b = w1_ref[_D_EMB:, :]
    ab = (
        jnp.dot(embed_ref[...], w1a, preferred_element_type=jnp.float32)
        + b1_ref[...]
    )
    bb = jnp.dot(embed_ref[...], w1b, preferred_element_type=jnp.float32)
    ab_ref[...] = ab[:_SC_ROWS, :]
    bb_ref[...] = bb[:_SC_ROWS, :]
    abt_ref[...] = ab.T
    bbt_ref[...] = bb.T
    aux_ref[0:1, :] = jnp.full((1, _L), 1.0, jnp.float32) / tmp_ref[0, 0]
    aux_ref[1:2, :] = jnp.full((1, _L), 1.0, jnp.float32) * b2_ref[0, 0]


def _prep(embed, W1, b1r, b2r, tmpr):
    return pl.pallas_call(
        _prep_body,
        out_shape=[
            jax.ShapeDtypeStruct((_SC_ROWS, _D_HID), jnp.float32),
            jax.ShapeDtypeStruct((_SC_ROWS, _D_HID), jnp.float32),
            jax.ShapeDtypeStruct((_D_HID, _N), jnp.float32),
            jax.ShapeDtypeStruct((_D_HID, _N), jnp.float32),
            jax.ShapeDtypeStruct((2, _L), jnp.float32),
        ],
    )(embed, W1, b1r, b2r, tmpr)


@functools.partial(
    pl.kernel,
    out_type=jax.ShapeDtypeStruct((_SC_ROWS, _N), jnp.float32),
    mesh=plsc.VectorSubcoreMesh(core_axis_name="c", subcore_axis_name="s"),
    compiler_params=pltpu.CompilerParams(needs_layout_passes=False),
    scratch_types=[
        pltpu.VMEM((_RW, _D_HID), jnp.float32),
        pltpu.VMEM((_RW, _D_HID), jnp.float32),
        pltpu.VMEM((_D_HID, _N), jnp.float32),
        pltpu.VMEM((_D_HID, _N), jnp.float32),
        pltpu.VMEM((_D_HID,), jnp.float32),
        pltpu.VMEM((2, _L), jnp.float32),
        pltpu.VMEM((_RW, _N), jnp.float32),
        pltpu.VMEM((_RW, _N), jnp.float32),
        pltpu.VMEM((_RW, _N), jnp.float32),
        pltpu.VMEM((_RW, _N), jnp.float32),
        pltpu.VMEM((_RW, _N), jnp.float32),
    ],
)
def _sc_pair(
    ab_hbm, bb_hbm, abt_hbm, bbt_hbm, w2_hbm, aux_hbm,
    adj_hbm, nl_hbm, nlt_hbm, out_hbm,
    ab_v, bb_v, abt_v, bbt_v, w2_v, aux_v,
    adj_v, nl_v, nlt_v, v1_v, v2_v,
):
    wid = lax.axis_index("s") * _NC + lax.axis_index("c")
    base = wid * _RW

    pltpu.sync_copy(ab_hbm.at[pl.ds(base, _RW)], ab_v)
    pltpu.sync_copy(bb_hbm.at[pl.ds(base, _RW)], bb_v)
    pltpu.sync_copy(abt_hbm, abt_v)
    pltpu.sync_copy(bbt_hbm, bbt_v)
    pltpu.sync_copy(w2_hbm, w2_v)
    pltpu.sync_copy(aux_hbm, aux_v)
    pltpu.sync_copy(adj_hbm.at[pl.ds(base, _RW)], adj_v)
    pltpu.sync_copy(nl_hbm.at[pl.ds(base, _RW)], nl_v)
    pltpu.sync_copy(nlt_hbm.at[pl.ds(base, _RW)], nlt_v)

    zero = jnp.zeros((_L,), jnp.float32)

    def zbody(r, c):
        for jc in range(_NJC):
            v1_v[r, pl.ds(jc * _L, _L)] = zero
            v2_v[r, pl.ds(jc * _L, _L)] = zero
        return c

    lax.fori_loop(0, _RW, zbody, 0)

    def accumulate(bt_ref, a_ref, acc_ref):
        def kbody(k, c):
            kv = jnp.full((_L,), k, jnp.int32)
            sw = plsc.load_gather(w2_v, [kv])
            bt = [bt_ref[k, pl.ds(jc * _L, _L)] for jc in range(_NJC)]

            def rbody(r, c2):
                rv = jnp.full((_L,), r, jnp.int32)
                sa = plsc.load_gather(a_ref, [rv, kv])
                for jc in range(_NJC):
                    t = jnp.maximum(bt[jc] + sa, 0.0) * sw
                    plsc.addupdate(acc_ref.at[r, pl.ds(jc * _L, _L)], t)
                return c2

            return lax.fori_loop(0, _RW, rbody, c)

        lax.fori_loop(0, _D_HID, kbody, 0)

    accumulate(bbt_v, ab_v, v1_v)
    accumulate(abt_v, bb_v, v2_v)

    itmp = aux_v[0, pl.ds(0, _L)]
    b2v = aux_v[1, pl.ds(0, _L)]
    half = jnp.full((_L,), 0.5, jnp.float32)
    one = jnp.full((_L,), 1.0, jnp.float32)

    def ebody(r, c):
        for jc in range(_NJC):
            d = pl.ds(jc * _L, _L)
            x1 = (nl_v[r, d] + v1_v[r, d] + b2v) * itmp
            g1 = one / (one + jnp.exp(-x1))
            x2 = (nlt_v[r, d] + v2_v[r, d] + b2v) * itmp
            g2 = one / (one + jnp.exp(-x2))
            v1_v[r, d] = adj_v[r, d] * (half * (g1 + g2))
        return c

    lax.fori_loop(0, _RW, ebody, 0)

    pltpu.sync_copy(v1_v, out_hbm.at[pl.ds(base, _RW)])


def _tc_pair_body(
    embed_ref, eblk_ref, w1_ref, b1_ref, w2_ref, b2_ref, tmp_ref,
    adj_ref, nl_ref, nlt_ref, out_ref, ba_scr,
):
    i = pl.program_id(0)
    w1a = w1_ref[:_D_EMB, :]
    w1b = w1_ref[_D_EMB:, :]

    @pl.when(i == 0)
    def _init():
        a_full = (
            jnp.dot(embed_ref[...], w1a, preferred_element_type=jnp.float32)
            + b1_ref[...]
        )
        b_full = jnp.dot(embed_ref[...], w1b, preferred_element_type=jnp.float32)
        ba_scr[...] = jnp.concatenate([b_full, a_full], axis=1).T

    a_i = (
        jnp.dot(eblk_ref[...], w1a, preferred_element_type=jnp.float32)
        + b1_ref[...]
    )
    b_i = jnp.dot(eblk_ref[...], w1b, preferred_element_type=jnp.float32)
    ab_i = jnp.concatenate([a_i, b_i], axis=1)

    itmp = 1.0 / tmp_ref[0, 0]
    b2 = b2_ref[0, 0]

    t12 = jnp.maximum(ab_i[:, :, None] + ba_scr[...][None, :, :], 0.0)
    w2full = jnp.concatenate([w2_ref[...], w2_ref[...]], axis=0)
    m = t12 * w2full.reshape(1, 2 * _D_HID, 1)
    v1 = jnp.sum(m[:, :_D_HID, :], axis=1)
    v2 = jnp.sum(m[:, _D_HID:, :], axis=1)

    g1 = jax.nn.sigmoid((nl_ref[...] + v1 + b2) * itmp)
    g2 = jax.nn.sigmoid((nlt_ref[...] + v2 + b2) * itmp)
    out_ref[...] = adj_ref[...] * (0.5 * (g1 + g2))


def _tc_pair(embed, W1, b1r, W2, b2r, tmpr, adj, nl, nlt):
    grid = (_TC_ROWS // _BI,)
    return pl.pallas_call(
        _tc_pair_body,
        grid=grid,
        in_specs=[
            pl.BlockSpec((_N, _D_EMB), lambda i: (0, 0)),
            pl.BlockSpec((_BI, _D_EMB), lambda i: (i + _TC_ROW0, 0)),
            pl.BlockSpec((2 * _D_EMB, _D_HID), lambda i: (0, 0)),
            pl.BlockSpec((1, _D_HID), lambda i: (0, 0)),
            pl.BlockSpec((_D_HID, 1), lambda i: (0, 0)),
            pl.BlockSpec((1, 1), lambda i: (0, 0)),
            pl.BlockSpec((1, 1), lambda i: (0, 0)),
            pl.BlockSpec((_BI, _N), lambda i: (i + _TC_ROW0, 0)),
            pl.BlockSpec((_BI, _N), lambda i: (i + _TC_ROW0, 0)),
            pl.BlockSpec((_BI, _N), lambda i: (i + _TC_ROW0, 0)),
        ],
        out_specs=pl.BlockSpec((_BI, _N), lambda i: (i + _TC_ROW0, 0)),
        out_shape=jax.ShapeDtypeStruct((_N, _N), jnp.float32),
        scratch_shapes=[
            pltpu.VMEM((2 * _D_HID, _N), jnp.float32),
        ],
    )(embed, embed, W1, b1r, W2, b2r, tmpr, adj, nl, nlt)


def kernel(x, embed, adj, W1, b1, W2, b2, tmp, label, sub_nodes):
    del x, label, sub_nodes
    nl_np, nlt_np = _noise_logit_np()
    nl = jnp.asarray(nl_np)
    nlt = jnp.asarray(nlt_np)
    b1r = b1.reshape(1, _D_HID)
    b2r = jnp.asarray(b2, jnp.float32).reshape(1, 1)
    tmpr = jnp.asarray(tmp, jnp.float32).reshape(1, 1)

    nl_sc = jnp.asarray(nl_np[:_SC_ROWS])
    nlt_sc = jnp.asarray(nlt_np[:_SC_ROWS])

    ab, bb, abt, bbt, aux = _prep(embed, W1, b1r, b2r, tmpr)

    w2f = W2.reshape(_D_HID)

    out_sc = _sc_pair(ab, bb, abt, bbt, w2f, aux, adj, nl_sc, nlt_sc)
    out_tc = _tc_pair(embed, W1, b1r, W2, b2r, tmpr, adj, nl, nlt)
    return lax.dynamic_update_slice(out_tc, out_sc, (0, 0))
